# baseline (speedup 1.0000x reference)
import functools

import jax
import jax.numpy as jnp
from jax import lax
from jax.experimental import pallas as pl
from jax.experimental.pallas import tpu as pltpu
from jax.experimental.pallas import tpu_sc as plsc

N_NODES = 100000
N_EDGES = 6400000
CHUNK = 2048
IDXR = CHUNK // 128
FR = CHUNK * 3 // 128
NCHUNKS = N_EDGES // CHUNK
NGROUPS = 10
TRIPS = (NCHUNKS + NGROUPS - 1) // NGROUPS
N_PAD = 100096
NR = N_PAD // 128


def _sc_scatter_partials(edge_index3, forces2, zeros):
    mesh = plsc.VectorSubcoreMesh(core_axis_name="c", subcore_axis_name="s")

    @functools.partial(
        pl.kernel,
        out_type=jax.ShapeDtypeStruct((2, 16, NR, 128), jnp.float32),
        mesh=mesh,
        compiler_params=pltpu.CompilerParams(
            use_tc_tiling_on_sc=False, needs_layout_passes=False),
        scratch_types=[
            pltpu.VMEM((NR, 128), jnp.float32),
            pltpu.VMEM((IDXR, 128), jnp.int32),
            pltpu.VMEM((IDXR, 128), jnp.int32),
            pltpu.VMEM((FR, 128), jnp.float32),
            pltpu.VMEM((FR, 128), jnp.float32),
            pltpu.SemaphoreType.DMA,
            pltpu.SemaphoreType.DMA,
            pltpu.SemaphoreType.DMA,
            pltpu.SemaphoreType.DMA,
        ],
    )
    def scatter_kernel(idx_hbm, f_hbm, z_hbm, part_hbm,
                       acc, i0, i1, f0, f1, si0, si1, sf0, sf1):
        core = lax.axis_index("c")
        s = lax.axis_index("s")
        group = s // 3
        comp = s % 3
        gid = core * 5 + group
        active = s < 15

        @pl.when(active)
        def _():
            pltpu.sync_copy(z_hbm, acc)

        bufs = ((i0, f0, si0, sf0), (i1, f1, si1, sf1))

        def issue_loads(t, ib, fb, si, sf):
            cid = gid + NGROUPS * t

            @pl.when(active & (cid < NCHUNKS))
            def _():
                pltpu.async_copy(
                    idx_hbm.at[0, pl.ds(cid * IDXR, IDXR), :], ib, si)
                pltpu.async_copy(
                    f_hbm.at[pl.ds(cid * FR, FR), :], fb, sf)

        for b in range(2):
            issue_loads(b, *bufs[b])

        iota3 = lax.iota(jnp.int32, 16) * 3

        def outer(o, _):
            for b in range(2):
                t = 2 * o + b
                ib, fb, si, sf = bufs[b]
                cid = gid + NGROUPS * t

                @pl.when(active & (cid < NCHUNKS))
                def _():
                    pltpu.make_async_copy(
                        idx_hbm.at[0, pl.ds(cid * IDXR, IDXR), :],
                        ib, si).wait()
                    pltpu.make_async_copy(
                        f_hbm.at[pl.ds(cid * FR, FR), :],
                        fb, sf).wait()

                    def body(q2, _):
                        for l in range(8):
                            nidx = ib[q2, pl.ds(l * 16, 16)]
                            p = iota3 + (q2 * 384 + l * 48 + comp)
                            vals = plsc.load_gather(
                                fb, [p >> 7, p & 127])
                            plsc.addupdate_scatter(
                                acc, [nidx >> 7, nidx & 127], vals)
                        return 0

                    lax.fori_loop(0, IDXR, body, 0)

                issue_loads(t + 2, ib, fb, si, sf)
            return 0

        lax.fori_loop(0, (TRIPS + 2) // 2, outer, 0)

        @pl.when(active)
        def _():
            pltpu.sync_copy(acc, part_hbm.at[core, s])

    return scatter_kernel(edge_index3, forces2, zeros)


def _tc_add_partials(parts):
    def add_body(p_ref, o_ref):
        for comp in range(3):
            rows = [16 * core + 3 * g + comp
                    for core in range(2) for g in range(5)]
            total = p_ref[rows[0]]
            for r in rows[1:]:
                total = total + p_ref[r]
            o_ref[comp, :, :] = total

    return pl.pallas_call(
        add_body,
        out_shape=jax.ShapeDtypeStruct((3, NR, 128), jnp.float32),
    )(parts)


def kernel(edge_index, edge_forces, atom_types):
    del atom_types
    zeros = jnp.zeros((NR, 128), jnp.float32)
    ei3 = edge_index.reshape(2, N_EDGES // 128, 128)
    ff = edge_forces.reshape(N_EDGES * 3 // 128, 128)
    parts = _sc_scatter_partials(ei3, ff, zeros)
    parts2 = parts.reshape(2 * 16, NR, 128)
    summed = _tc_add_partials(parts2)
    out3n = summed.reshape(3, N_PAD)[:, :N_NODES]
    return out3n.T


# speedup vs baseline: 24.0562x; 24.0562x over previous
import functools

import jax
import jax.numpy as jnp
from jax import lax
from jax.experimental import pallas as pl
from jax.experimental.pallas import tpu as pltpu
from jax.experimental.pallas import tpu_sc as plsc

N_NODES = 100000
N_EDGES = 6400000
CHUNK = 2048
IDXR = CHUNK // 128
NCHUNKS = N_EDGES // CHUNK
NGROUPS = 10
TRIPS = (NCHUNKS + NGROUPS - 1) // NGROUPS
N_PAD = 100096
NR = N_PAD // 128


def _sc_scatter_partials(eidx, fpad, zeros):
    mesh = plsc.VectorSubcoreMesh(core_axis_name="c", subcore_axis_name="s")

    @functools.partial(
        pl.kernel,
        out_type=jax.ShapeDtypeStruct((2, 16, NR, 128), jnp.float32),
        mesh=mesh,
        compiler_params=pltpu.CompilerParams(
            use_tc_tiling_on_sc=False, needs_layout_passes=False),
        scratch_types=[
            pltpu.VMEM((NR, 128), jnp.float32),
            pltpu.VMEM((IDXR, 128), jnp.int32),
            pltpu.VMEM((IDXR, 128), jnp.int32),
            pltpu.VMEM((IDXR, 128), jnp.float32),
            pltpu.VMEM((IDXR, 128), jnp.float32),
            pltpu.SemaphoreType.DMA,
            pltpu.SemaphoreType.DMA,
            pltpu.SemaphoreType.DMA,
            pltpu.SemaphoreType.DMA,
        ],
    )
    def scatter_kernel(idx_hbm, f_hbm, z_hbm, part_hbm,
                       acc, i0, i1, f0, f1, si0, si1, sf0, sf1):
        core = lax.axis_index("c")
        s = lax.axis_index("s")
        group = s // 3
        comp = s % 3
        gid = core * 5 + group
        active = s < 15

        @pl.when(active)
        def _():
            pltpu.sync_copy(z_hbm, acc)

        bufs = ((i0, f0, si0, sf0), (i1, f1, si1, sf1))

        def issue_loads(t, ib, fb, si, sf):
            cid = gid + NGROUPS * t

            @pl.when(active & (cid < NCHUNKS))
            def _():
                pltpu.async_copy(
                    idx_hbm.at[pl.ds(cid * IDXR, IDXR), 0, :], ib, si)
                pltpu.async_copy(
                    f_hbm.at[pl.ds(cid * IDXR, IDXR), comp, :], fb, sf)

        for b in range(2):
            issue_loads(b, *bufs[b])

        def outer(o, _):
            for b in range(2):
                t = 2 * o + b
                ib, fb, si, sf = bufs[b]
                cid = gid + NGROUPS * t

                @pl.when(active & (cid < NCHUNKS))
                def _():
                    pltpu.make_async_copy(
                        idx_hbm.at[pl.ds(cid * IDXR, IDXR), 0, :],
                        ib, si).wait()
                    pltpu.make_async_copy(
                        f_hbm.at[pl.ds(cid * IDXR, IDXR), comp, :],
                        fb, sf).wait()

                    def body(q2, _):
                        for l in range(8):
                            nidx = ib[q2, pl.ds(l * 16, 16)]
                            vals = fb[q2, pl.ds(l * 16, 16)]
                            plsc.addupdate_scatter(
                                acc, [nidx >> 7, nidx & 127], vals)
                        return 0

                    lax.fori_loop(0, IDXR, body, 0)

                issue_loads(t + 2, ib, fb, si, sf)
            return 0

        lax.fori_loop(0, (TRIPS + 2) // 2, outer, 0)

        @pl.when(active)
        def _():
            pltpu.sync_copy(acc, part_hbm.at[core, s])

    return scatter_kernel(eidx, fpad, zeros)


def _tc_add_partials(parts):
    def add_body(p_ref, o_ref):
        for comp in range(3):
            rows = [16 * core + 3 * g + comp
                    for core in range(2) for g in range(5)]
            total = p_ref[rows[0]]
            for r in rows[1:]:
                total = total + p_ref[r]
            o_ref[comp, :, :] = total

    return pl.pallas_call(
        add_body,
        out_shape=jax.ShapeDtypeStruct((3, NR, 128), jnp.float32),
    )(parts)


def kernel(edge_index, edge_forces, atom_types):
    del atom_types
    zeros = jnp.zeros((NR, 128), jnp.float32)
    eidx = edge_index.reshape(2, N_EDGES // 128, 128).transpose(1, 0, 2)
    ep = edge_forces.reshape(N_EDGES // 128, 128, 3).transpose(0, 2, 1)
    fpad = jnp.concatenate(
        [ep, jnp.zeros((N_EDGES // 128, 1, 128), jnp.float32)], axis=1)
    parts = _sc_scatter_partials(eidx, fpad, zeros)
    parts2 = parts.reshape(2 * 16, NR, 128)
    summed = _tc_add_partials(parts2)
    out3n = summed.reshape(3, N_PAD)[:, :N_NODES]
    return out3n.T


# speedup vs baseline: 24.3088x; 1.0105x over previous
import functools

import jax
import jax.numpy as jnp
from jax import lax
from jax.experimental import pallas as pl
from jax.experimental.pallas import tpu as pltpu
from jax.experimental.pallas import tpu_sc as plsc

N_NODES = 100000
N_EDGES = 6400000
CHUNK = 2048
IDXR = CHUNK // 128
NCHUNKS = N_EDGES // CHUNK
NGROUPS = 10
TRIPS = (NCHUNKS + NGROUPS - 1) // NGROUPS
N_PAD = 100096
NR = N_PAD // 128


def _sc_scatter_partials(eidx, fpad, zeros):
    mesh = plsc.VectorSubcoreMesh(core_axis_name="c", subcore_axis_name="s")

    @functools.partial(
        pl.kernel,
        out_type=jax.ShapeDtypeStruct((2, 16, N_PAD), jnp.float32),
        mesh=mesh,
        compiler_params=pltpu.CompilerParams(
            use_tc_tiling_on_sc=False, needs_layout_passes=False),
        scratch_types=[
            pltpu.VMEM((N_PAD,), jnp.float32),
            pltpu.VMEM((IDXR, 128), jnp.int32),
            pltpu.VMEM((IDXR, 128), jnp.int32),
            pltpu.VMEM((IDXR, 128), jnp.float32),
            pltpu.VMEM((IDXR, 128), jnp.float32),
            pltpu.SemaphoreType.DMA,
            pltpu.SemaphoreType.DMA,
            pltpu.SemaphoreType.DMA,
            pltpu.SemaphoreType.DMA,
        ],
    )
    def scatter_kernel(idx_hbm, f_hbm, z_hbm, part_hbm,
                       acc, i0, i1, f0, f1, si0, si1, sf0, sf1):
        core = lax.axis_index("c")
        s = lax.axis_index("s")
        group = s // 3
        comp = s % 3
        gid = core * 5 + group
        active = s < 15

        @pl.when(active)
        def _():
            pltpu.sync_copy(z_hbm, acc)

        bufs = ((i0, f0, si0, sf0), (i1, f1, si1, sf1))

        def issue_loads(t, ib, fb, si, sf):
            cid = gid + NGROUPS * t

            @pl.when(active & (cid < NCHUNKS))
            def _():
                pltpu.async_copy(
                    idx_hbm.at[pl.ds(cid * IDXR, IDXR), 0, :], ib, si)
                pltpu.async_copy(
                    f_hbm.at[pl.ds(cid * IDXR, IDXR), comp, :], fb, sf)

        for b in range(2):
            issue_loads(b, *bufs[b])

        def outer(o, _):
            for b in range(2):
                t = 2 * o + b
                ib, fb, si, sf = bufs[b]
                cid = gid + NGROUPS * t

                @pl.when(active & (cid < NCHUNKS))
                def _():
                    pltpu.make_async_copy(
                        idx_hbm.at[pl.ds(cid * IDXR, IDXR), 0, :],
                        ib, si).wait()
                    pltpu.make_async_copy(
                        f_hbm.at[pl.ds(cid * IDXR, IDXR), comp, :],
                        fb, sf).wait()

                    for q2 in range(IDXR):
                        for l in range(8):
                            nidx = ib[q2, pl.ds(l * 16, 16)]
                            vals = fb[q2, pl.ds(l * 16, 16)]
                            plsc.addupdate_scatter(acc, [nidx], vals)

                issue_loads(t + 2, ib, fb, si, sf)
            return 0

        lax.fori_loop(0, (TRIPS + 2) // 2, outer, 0)

        @pl.when(active)
        def _():
            pltpu.sync_copy(acc, part_hbm.at[core, s])

    return scatter_kernel(eidx, fpad, zeros)


def _tc_add_partials(parts):
    def add_body(p_ref, o_ref):
        for comp in range(3):
            rows = [16 * core + 3 * g + comp
                    for core in range(2) for g in range(5)]
            total = p_ref[rows[0]]
            for r in rows[1:]:
                total = total + p_ref[r]
            o_ref[comp, :, :] = total

    return pl.pallas_call(
        add_body,
        out_shape=jax.ShapeDtypeStruct((3, NR, 128), jnp.float32),
    )(parts)


def kernel(edge_index, edge_forces, atom_types):
    del atom_types
    zeros = jnp.zeros((N_PAD,), jnp.float32)
    eidx = edge_index.reshape(2, N_EDGES // 128, 128).transpose(1, 0, 2)
    ep = edge_forces.reshape(N_EDGES // 128, 128, 3).transpose(0, 2, 1)
    fpad = jnp.concatenate(
        [ep, jnp.zeros((N_EDGES // 128, 1, 128), jnp.float32)], axis=1)
    parts = _sc_scatter_partials(eidx, fpad, zeros)
    parts2 = parts.reshape(2 * 16, NR, 128)
    summed = _tc_add_partials(parts2)
    out3n = summed.reshape(3, N_PAD)[:, :N_NODES]
    return out3n.T
